# baseline (speedup 1.0000x reference)
import functools

import jax
import jax.numpy as jnp
from jax import lax
from jax.experimental import pallas as pl
from jax.experimental.pallas import tpu as pltpu
from jax.experimental.pallas import tpu_sc as plsc

NN = 10000
NE = 320000
H = 32
TW = 128
NC, NS = 2, 16
NW = NC * NS
CH = 64
NCHUNK = NE // CH
KMAX = -(-NCHUNK // NW)
NNP = 10240
RPT = NNP // NS

_f32 = jnp.float32


def _norm_body(e_ref, o_ref):
  o_ref[...] = jnp.max(jnp.abs(e_ref[...])).reshape(1, 1)


def _encode_body(n_ref, wne_ref, bne_ref, ws_ref, wr_ref, hn_ref, t_ref):
  hn = jnp.maximum(
      jnp.dot(n_ref[...], wne_ref[...], preferred_element_type=_f32)
      + bne_ref[...], 0.0)
  hn_ref[...] = hn
  a = jnp.dot(hn, ws_ref[...], preferred_element_type=_f32)
  b = jnp.dot(hn, wr_ref[...], preferred_element_type=_f32)
  z = jnp.zeros((a.shape[0], TW - 2 * H), _f32)
  t_ref[...] = jnp.concatenate([a, b, z], axis=1)


def _c0_body(e_ref, nrm_ref, wee_ref, bee_ref, wse_ref, be_ref, c_ref):
  inv = 1.0 / nrm_ref[0, 0]
  he = jnp.maximum(e_ref[...] * inv * wee_ref[...] + bee_ref[...], 0.0)
  c_ref[...] = jnp.dot(he, wse_ref[...], preferred_element_type=_f32) + be_ref[...]


def _node_body(hn_ref, p0_ref, p1_ref, wn1_ref, wn2_ref, bn_ref, ws_ref, wr_ref,
               hn2_ref, t_ref):
  agg = p0_ref[0, :, :H] + p1_ref[0, :, :H]
  hn2 = jnp.maximum(
      jnp.dot(hn_ref[...], wn1_ref[...], preferred_element_type=_f32)
      + jnp.dot(agg, wn2_ref[...], preferred_element_type=_f32)
      + bn_ref[...], 0.0)
  hn2_ref[...] = hn2
  a = jnp.dot(hn2, ws_ref[...], preferred_element_type=_f32)
  b = jnp.dot(hn2, wr_ref[...], preferred_element_type=_f32)
  z = jnp.zeros((a.shape[0], TW - 2 * H), _f32)
  t_ref[...] = jnp.concatenate([a, b, z], axis=1)


def _cmat_body(m_ref, wse_ref, be_ref, c_ref):
  c_ref[...] = (jnp.dot(m_ref[...], wse_ref[...], preferred_element_type=_f32)
                + be_ref[...])


def _decmask_body(m_ref, wed_ref, bed_ref, e_ref, r_ref, s_ref, nrm_ref,
                  al_ref, o_ref):
  dec = (jnp.dot(m_ref[...], wed_ref[...], preferred_element_type=_f32)
         + bed_ref[...])
  scale = al_ref[0, 0] * nrm_ref[0, 0]
  val = e_ref[...] + scale * dec
  o_ref[...] = jnp.where(r_ref[...] >= s_ref[...], val, 0.0)


def _full(shape):
  return pl.BlockSpec(shape, lambda *a: (0,) * len(shape))


def _sc_layer_body(with_agg, t_hbm, c_hbm, s_hbm, r_hbm, *rest):
  if with_agg:
    (m_hbm, aggp_hbm, sidx0, sidx1, ridx0, ridx1, ga, gb, rc0, rc1, mb,
     agg_sh, semA, semB, semC, semI, semM, semS) = rest
  else:
    (m_hbm, sidx0, sidx1, ridx0, ridx1, ga, gb, rc0, rc1,
     semA, semB, semC, semI, semM) = rest
  sidx = (sidx0, sidx1)
  ridx = (ridx0, ridx1)
  rc = (rc0, rc1)

  cid = lax.axis_index("c")
  sid = lax.axis_index("s")
  wid = sid * NC + cid

  if with_agg:
    def zloop(j, _):
      for h in range(TW // 16):
        mb[j, pl.ds(16 * h, 16)] = jnp.zeros((16,), _f32)
      return 0
    lax.fori_loop(0, CH, zloop, 0)

    def zcopy(t, _):
      pltpu.sync_copy(mb, agg_sh.at[pl.ds(sid * RPT + t * CH, CH)])
      return 0
    lax.fori_loop(0, RPT // CH, zcopy, 0)
    plsc.subcore_barrier()

  def issue_fetch(g, p):
    base = g * CH
    ci0 = pltpu.async_copy(s_hbm.at[pl.ds(base, CH)], sidx[p], semI)
    ci1 = pltpu.async_copy(r_hbm.at[pl.ds(base, CH)], ridx[p], semI)
    ci0.wait()
    ci1.wait()
    pltpu.async_copy(t_hbm.at[sidx[p]], ga, semA)
    pltpu.async_copy(t_hbm.at[ridx[p]], gb, semB)
    pltpu.async_copy(c_hbm.at[pl.ds(base, CH)], rc[p], semC)

  def wait_gathers(p):
    pltpu.make_async_copy(t_hbm.at[sidx[p]], ga, semA).wait()
    pltpu.make_async_copy(t_hbm.at[ridx[p]], gb, semB).wait()
    pltpu.make_async_copy(c_hbm.at[pl.ds(0, CH)], rc[p], semC).wait()

  issue_fetch(wid, 0)

  def body(k, p):
    g = k * NW + wid

    @pl.when(g < NCHUNK)
    def _():
      base = g * CH
      wait_gathers(p)
      if with_agg:
        @pl.when(k > 0)
        def _():
          pltpu.make_async_copy(mb, agg_sh.at[ridx[p]], semS).wait()

      @plsc.parallel_loop(0, CH, 1, unroll=8)
      def comp(j):
        for h in (0, 16):
          v = (ga[j, pl.ds(h, 16)] + gb[j, pl.ds(H + h, 16)]
               + rc[p][j, pl.ds(h, 16)])
          v = jnp.maximum(v, 0.0)
          rc[p][j, pl.ds(h, 16)] = v
          if with_agg:
            mb[j, pl.ds(h, 16)] = v

      pltpu.async_copy(rc[p], m_hbm.at[pl.ds(base, CH)], semM)
      if with_agg:
        pltpu.async_copy(mb, agg_sh.at[ridx[p]], semS, add=True)

      g1 = g + NW

      @pl.when(g1 < NCHUNK)
      def _():
        @pl.when(k > 0)
        def _():
          pltpu.make_async_copy(rc[1 - p], m_hbm.at[pl.ds(0, CH)], semM).wait()
        issue_fetch(g1, 1 - p)

  def two(kk, _):
    body(2 * kk, 0)
    body(2 * kk + 1, 1)
    return 0
  lax.fori_loop(0, (KMAX + 1) // 2, two, 0)

  pltpu.make_async_copy(rc0, m_hbm.at[pl.ds(0, CH)], semM).wait()
  pltpu.make_async_copy(rc0, m_hbm.at[pl.ds(0, CH)], semM).wait()
  if with_agg:
    pltpu.make_async_copy(mb, agg_sh.at[ridx0], semS).wait()
    plsc.subcore_barrier()
    r0 = sid * RPT
    pltpu.sync_copy(agg_sh.at[pl.ds(r0, RPT)], aggp_hbm.at[cid, pl.ds(r0, RPT)])


def _make_sc_layer(with_agg):
  mesh = plsc.VectorSubcoreMesh(
      core_axis_name="c", subcore_axis_name="s", num_cores=NC, num_subcores=NS)
  if with_agg:
    out_type = (jax.ShapeDtypeStruct((NE, H), _f32),
                jax.ShapeDtypeStruct((NC, NNP, TW), _f32))
  else:
    out_type = jax.ShapeDtypeStruct((NE, H), _f32)
  scratch = [
      pltpu.VMEM((CH,), jnp.int32),
      pltpu.VMEM((CH,), jnp.int32),
      pltpu.VMEM((CH,), jnp.int32),
      pltpu.VMEM((CH,), jnp.int32),
      pltpu.VMEM((CH, TW), _f32),
      pltpu.VMEM((CH, TW), _f32),
      pltpu.VMEM((CH, H), _f32),
      pltpu.VMEM((CH, H), _f32),
  ]
  if with_agg:
    scratch.append(pltpu.VMEM((CH, TW), _f32))
    scratch.append(pltpu.VMEM_SHARED((NNP, TW), _f32))
  nsem = 6 if with_agg else 5
  scratch += [pltpu.SemaphoreType.DMA] * nsem
  return pl.kernel(
      functools.partial(_sc_layer_body, with_agg),
      out_type=out_type, mesh=mesh, scratch_types=scratch,
      name="sc_layer_agg" if with_agg else "sc_layer")


_sc_layer_agg = _make_sc_layer(True)
_sc_layer_noagg = _make_sc_layer(False)


def kernel(nodes, edges, receivers, senders, Wne, bne, Wee, bee, We, be, Wn,
           bn, Wed, bed, alpha):
  eflat = edges.reshape(NE)
  norm = pl.pallas_call(
      _norm_body,
      out_shape=jax.ShapeDtypeStruct((1, 1), _f32),
  )(eflat.reshape(2500, 128))

  NB = 1000
  hn, T = pl.pallas_call(
      _encode_body,
      grid=(NN // NB,),
      in_specs=[pl.BlockSpec((NB, 128), lambda i: (i, 0)),
                _full((128, H)), _full((1, H)), _full((H, H)), _full((H, H))],
      out_specs=[pl.BlockSpec((NB, H), lambda i: (i, 0)),
                 pl.BlockSpec((NB, TW), lambda i: (i, 0))],
      out_shape=[jax.ShapeDtypeStruct((NN, H), _f32),
                 jax.ShapeDtypeStruct((NN, TW), _f32)],
  )(nodes, Wne, bne.reshape(1, H), We[0, :H], We[0, H:2 * H])

  EB = 4000
  C = pl.pallas_call(
      _c0_body,
      grid=(NE // EB,),
      in_specs=[pl.BlockSpec((EB, 1), lambda i: (i, 0)),
                _full((1, 1)), _full((1, H)), _full((1, H)),
                _full((H, H)), _full((1, H))],
      out_specs=pl.BlockSpec((EB, H), lambda i: (i, 0)),
      out_shape=jax.ShapeDtypeStruct((NE, H), _f32),
  )(edges, norm, Wee, bee.reshape(1, H), We[0, 2 * H:], be[0].reshape(1, H))

  m, aggp = _sc_layer_agg(T, C, senders, receivers)

  for i in (1, 2):
    hn, T = pl.pallas_call(
        _node_body,
        grid=(NN // NB,),
        in_specs=[pl.BlockSpec((NB, H), lambda i: (i, 0)),
                  pl.BlockSpec((1, NB, TW), lambda i: (0, i, 0)),
                  pl.BlockSpec((1, NB, TW), lambda i: (1, i, 0)),
                  _full((H, H)), _full((H, H)), _full((1, H)),
                  _full((H, H)), _full((H, H))],
        out_specs=[pl.BlockSpec((NB, H), lambda i: (i, 0)),
                   pl.BlockSpec((NB, TW), lambda i: (i, 0))],
        out_shape=[jax.ShapeDtypeStruct((NN, H), _f32),
                   jax.ShapeDtypeStruct((NN, TW), _f32)],
    )(hn, aggp, aggp, Wn[i - 1, :H], Wn[i - 1, H:],
      bn[i - 1].reshape(1, H), We[i, :H], We[i, H:2 * H])

    C = pl.pallas_call(
        _cmat_body,
        grid=(NE // EB,),
        in_specs=[pl.BlockSpec((EB, H), lambda i: (i, 0)),
                  _full((H, H)), _full((1, H))],
        out_specs=pl.BlockSpec((EB, H), lambda i: (i, 0)),
        out_shape=jax.ShapeDtypeStruct((NE, H), _f32),
    )(m, We[i, 2 * H:], be[i].reshape(1, H))

    if i < 2:
      m, aggp = _sc_layer_agg(T, C, senders, receivers)
    else:
      m = _sc_layer_noagg(T, C, senders, receivers)

  out = pl.pallas_call(
      _decmask_body,
      grid=(NE // EB,),
      in_specs=[pl.BlockSpec((EB, H), lambda i: (i, 0)),
                _full((H, 1)), _full((1, 1)),
                pl.BlockSpec((EB, 1), lambda i: (i, 0)),
                pl.BlockSpec((EB, 1), lambda i: (i, 0)),
                pl.BlockSpec((EB, 1), lambda i: (i, 0)),
                _full((1, 1)), _full((1, 1))],
      out_specs=pl.BlockSpec((EB, 1), lambda i: (i, 0)),
      out_shape=jax.ShapeDtypeStruct((NE, 1), _f32),
  )(m, Wed, bed.reshape(1, 1), edges,
    receivers.reshape(NE, 1), senders.reshape(NE, 1),
    norm, alpha.reshape(1, 1))

  return out.reshape(NE)


# speedup vs baseline: 1.3659x; 1.3659x over previous
import functools

import jax
import jax.numpy as jnp
from jax import lax
from jax.experimental import pallas as pl
from jax.experimental.pallas import tpu as pltpu
from jax.experimental.pallas import tpu_sc as plsc

NN = 10000
NE = 320000
H = 32
TW = 128
NC, NS = 2, 16
NW = NC * NS
CH = 64
NCHUNK = NE // CH
KMAX = -(-NCHUNK // NW)
NNP = 10240
RPT = NNP // NS
EP = NE // 8
CHP = CH // 8

_f32 = jnp.float32


def _norm_body(e_ref, o_ref):
  o_ref[...] = jnp.max(jnp.abs(e_ref[...])).reshape(1, 1)


def _encode_body(n_ref, wne_ref, bne_ref, ws_ref, wr_ref, hn_ref, t_ref):
  hn = jnp.maximum(
      jnp.dot(n_ref[...], wne_ref[...], preferred_element_type=_f32)
      + bne_ref[...], 0.0)
  hn_ref[...] = hn
  a = jnp.dot(hn, ws_ref[...], preferred_element_type=_f32)
  b = jnp.dot(hn, wr_ref[...], preferred_element_type=_f32)
  z = jnp.zeros((a.shape[0], TW - 2 * H), _f32)
  t_ref[...] = jnp.concatenate([a, b, z], axis=1)


def _c0_body(e_ref, nrm_ref, wee_ref, bee_ref, wse_ref, be_ref, c_ref):
  inv = 1.0 / nrm_ref[0, 0]
  he = jnp.maximum(e_ref[...] * inv * wee_ref[...] + bee_ref[...], 0.0)
  c_ref[...] = jnp.dot(he, wse_ref[...], preferred_element_type=_f32) + be_ref[...]


def _node_body(hn_ref, p0_ref, p1_ref, wn1_ref, wn2_ref, bn_ref, ws_ref, wr_ref,
               hn2_ref, t_ref):
  agg = p0_ref[0, :, :H] + p1_ref[0, :, :H]
  hn2 = jnp.maximum(
      jnp.dot(hn_ref[...], wn1_ref[...], preferred_element_type=_f32)
      + jnp.dot(agg, wn2_ref[...], preferred_element_type=_f32)
      + bn_ref[...], 0.0)
  hn2_ref[...] = hn2
  a = jnp.dot(hn2, ws_ref[...], preferred_element_type=_f32)
  b = jnp.dot(hn2, wr_ref[...], preferred_element_type=_f32)
  z = jnp.zeros((a.shape[0], TW - 2 * H), _f32)
  t_ref[...] = jnp.concatenate([a, b, z], axis=1)


def _cmat_body(m_ref, wse_ref, be_ref, c_ref):
  c_ref[...] = (jnp.dot(m_ref[...], wse_ref[...], preferred_element_type=_f32)
                + be_ref[...])


def _dec_body(m_ref, wed_ref, bed_ref, d_ref):
  d_ref[...] = (jnp.dot(m_ref[...], wed_ref[...], preferred_element_type=_f32)
                + bed_ref[...])


def _mask_body(d_ref, e_ref, r_ref, s_ref, nrm_ref, al_ref, o_ref):
  scale = al_ref[0, 0] * nrm_ref[0, 0]
  val = e_ref[...] + scale * d_ref[...]
  o_ref[...] = jnp.where(r_ref[...] >= s_ref[...], val, 0.0)


def _full(shape):
  return pl.BlockSpec(shape, lambda *a: (0,) * len(shape))


def _sc_layer_body(with_agg, c_packed, t_hbm, c_hbm, s_hbm, r_hbm, *rest):
  if with_agg:
    (m_hbm, aggp_hbm, sidx0, sidx1, ridx0, ridx1, ga, gb, rc0, rc1,
     mc0, mc1, mb, agg_sh, semA, semB, semC, semI, semM, semS) = rest
  else:
    (m_hbm, sidx0, sidx1, ridx0, ridx1, ga, gb, rc0, rc1, mc0, mc1,
     semA, semB, semC, semI, semM) = rest
  sidx = (sidx0, sidx1)
  ridx = (ridx0, ridx1)
  rc = (rc0, rc1)
  mc = (mc0, mc1)

  cid = lax.axis_index("c")
  sid = lax.axis_index("s")
  wid = sid * NC + cid

  if with_agg:
    def zloop(j, _):
      for h in range(TW // 16):
        mb[j, pl.ds(16 * h, 16)] = jnp.zeros((16,), _f32)
      return 0
    lax.fori_loop(0, CH, zloop, 0)

    def zcopy(t, _):
      pltpu.sync_copy(mb, agg_sh.at[pl.ds(sid * RPT + t * CH, CH)])
      return 0
    lax.fori_loop(0, RPT // CH, zcopy, 0)
    plsc.subcore_barrier()

  def issue_fetch(g, p):
    base = g * CH
    ci0 = pltpu.async_copy(s_hbm.at[pl.ds(base, CH)], sidx[p], semI)
    ci1 = pltpu.async_copy(r_hbm.at[pl.ds(base, CH)], ridx[p], semI)
    ci0.wait()
    ci1.wait()
    pltpu.async_copy(t_hbm.at[sidx[p]], ga, semA)
    pltpu.async_copy(t_hbm.at[ridx[p]], gb, semB)
    if c_packed:
      pltpu.async_copy(c_hbm.at[pl.ds(g * CHP, CHP)], rc[p], semC)
    else:
      pltpu.async_copy(c_hbm.at[pl.ds(base, CH)], rc[p], semC)

  def wait_gathers(p):
    pltpu.make_async_copy(t_hbm.at[sidx[p]], ga, semA).wait()
    pltpu.make_async_copy(t_hbm.at[ridx[p]], gb, semB).wait()
    pltpu.make_async_copy(c_hbm.at[pl.ds(0, CHP if c_packed else CH)],
                          rc[p], semC).wait()

  issue_fetch(wid, 0)

  def body(k, p):
    g = k * NW + wid

    @pl.when(g < NCHUNK)
    def _():
      base = g * CH
      wait_gathers(p)
      if with_agg:
        @pl.when(k > 0)
        def _():
          pltpu.make_async_copy(mb, agg_sh.at[ridx[p]], semS).wait()

      @plsc.parallel_loop(0, CH, 1, unroll=8)
      def comp(j):
        jq = j // 8
        jo = (j % 8) * H
        for h in (0, 16):
          if c_packed:
            cv = rc[p][jq, pl.ds(pl.multiple_of(jo + h, 16), 16)]
          else:
            cv = rc[p][j, pl.ds(h, 16)]
          v = ga[j, pl.ds(h, 16)] + gb[j, pl.ds(H + h, 16)] + cv
          v = jnp.maximum(v, 0.0)
          mc[p][jq, pl.ds(pl.multiple_of(jo + h, 16), 16)] = v
          if with_agg:
            mb[j, pl.ds(h, 16)] = v

      pltpu.async_copy(mc[p], m_hbm.at[pl.ds(g * CHP, CHP)], semM)
      if with_agg:
        pltpu.async_copy(mb, agg_sh.at[ridx[p]], semS, add=True)

      g1 = g + NW

      @pl.when(g1 < NCHUNK)
      def _():
        @pl.when(k > 0)
        def _():
          pltpu.make_async_copy(mc[1 - p], m_hbm.at[pl.ds(0, CHP)], semM).wait()
        issue_fetch(g1, 1 - p)

  def two(kk, _):
    body(2 * kk, 0)
    body(2 * kk + 1, 1)
    return 0
  lax.fori_loop(0, (KMAX + 1) // 2, two, 0)

  pltpu.make_async_copy(mc0, m_hbm.at[pl.ds(0, CHP)], semM).wait()
  pltpu.make_async_copy(mc0, m_hbm.at[pl.ds(0, CHP)], semM).wait()
  if with_agg:
    pltpu.make_async_copy(mb, agg_sh.at[ridx0], semS).wait()
    plsc.subcore_barrier()
    r0 = sid * RPT
    pltpu.sync_copy(agg_sh.at[pl.ds(r0, RPT)], aggp_hbm.at[cid, pl.ds(r0, RPT)])


def _make_sc_layer(with_agg, c_packed):
  mesh = plsc.VectorSubcoreMesh(
      core_axis_name="c", subcore_axis_name="s", num_cores=NC, num_subcores=NS)
  if with_agg:
    out_type = (jax.ShapeDtypeStruct((EP, 8 * H), _f32),
                jax.ShapeDtypeStruct((NC, NNP, TW), _f32))
  else:
    out_type = jax.ShapeDtypeStruct((EP, 8 * H), _f32)
  cshape = (CHP, 8 * H) if c_packed else (CH, H)
  scratch = [
      pltpu.VMEM((CH,), jnp.int32),
      pltpu.VMEM((CH,), jnp.int32),
      pltpu.VMEM((CH,), jnp.int32),
      pltpu.VMEM((CH,), jnp.int32),
      pltpu.VMEM((CH, TW), _f32),
      pltpu.VMEM((CH, TW), _f32),
      pltpu.VMEM(cshape, _f32),
      pltpu.VMEM(cshape, _f32),
      pltpu.VMEM((CHP, 8 * H), _f32),
      pltpu.VMEM((CHP, 8 * H), _f32),
  ]
  if with_agg:
    scratch.append(pltpu.VMEM((CH, TW), _f32))
    scratch.append(pltpu.VMEM_SHARED((NNP, TW), _f32))
  nsem = 6 if with_agg else 5
  scratch += [pltpu.SemaphoreType.DMA] * nsem
  return pl.kernel(
      functools.partial(_sc_layer_body, with_agg, c_packed),
      out_type=out_type, mesh=mesh, scratch_types=scratch,
      name="sc_layer_agg" if with_agg else "sc_layer")


_sc_layer_agg0 = _make_sc_layer(True, False)
_sc_layer_agg = _make_sc_layer(True, True)
_sc_layer_noagg = _make_sc_layer(False, True)


def kernel(nodes, edges, receivers, senders, Wne, bne, Wee, bee, We, be, Wn,
           bn, Wed, bed, alpha):
  eflat = edges.reshape(NE)
  norm = pl.pallas_call(
      _norm_body,
      out_shape=jax.ShapeDtypeStruct((1, 1), _f32),
  )(eflat.reshape(2500, 128))

  NB = 1000
  hn, T = pl.pallas_call(
      _encode_body,
      grid=(NN // NB,),
      in_specs=[pl.BlockSpec((NB, 128), lambda i: (i, 0)),
                _full((128, H)), _full((1, H)), _full((H, H)), _full((H, H))],
      out_specs=[pl.BlockSpec((NB, H), lambda i: (i, 0)),
                 pl.BlockSpec((NB, TW), lambda i: (i, 0))],
      out_shape=[jax.ShapeDtypeStruct((NN, H), _f32),
                 jax.ShapeDtypeStruct((NN, TW), _f32)],
  )(nodes, Wne, bne.reshape(1, H), We[0, :H], We[0, H:2 * H])

  EB = 4000
  C = pl.pallas_call(
      _c0_body,
      grid=(NE // EB,),
      in_specs=[pl.BlockSpec((EB, 1), lambda i: (i, 0)),
                _full((1, 1)), _full((1, H)), _full((1, H)),
                _full((H, H)), _full((1, H))],
      out_specs=pl.BlockSpec((EB, H), lambda i: (i, 0)),
      out_shape=jax.ShapeDtypeStruct((NE, H), _f32),
  )(edges, norm, Wee, bee.reshape(1, H), We[0, 2 * H:], be[0].reshape(1, H))

  m, aggp = _sc_layer_agg0(T, C, senders, receivers)

  eye8 = jnp.eye(8, dtype=_f32)
  for i in (1, 2):
    hn, T = pl.pallas_call(
        _node_body,
        grid=(NN // NB,),
        in_specs=[pl.BlockSpec((NB, H), lambda i: (i, 0)),
                  pl.BlockSpec((1, NB, TW), lambda i: (0, i, 0)),
                  pl.BlockSpec((1, NB, TW), lambda i: (1, i, 0)),
                  _full((H, H)), _full((H, H)), _full((1, H)),
                  _full((H, H)), _full((H, H))],
        out_specs=[pl.BlockSpec((NB, H), lambda i: (i, 0)),
                   pl.BlockSpec((NB, TW), lambda i: (i, 0))],
        out_shape=[jax.ShapeDtypeStruct((NN, H), _f32),
                   jax.ShapeDtypeStruct((NN, TW), _f32)],
    )(hn, aggp, aggp, Wn[i - 1, :H], Wn[i - 1, H:],
      bn[i - 1].reshape(1, H), We[i, :H], We[i, H:2 * H])

    C = pl.pallas_call(
        _cmat_body,
        grid=(EP // 1000,),
        in_specs=[pl.BlockSpec((1000, 8 * H), lambda i: (i, 0)),
                  _full((8 * H, 8 * H)), _full((1, 8 * H))],
        out_specs=pl.BlockSpec((1000, 8 * H), lambda i: (i, 0)),
        out_shape=jax.ShapeDtypeStruct((EP, 8 * H), _f32),
    )(m, jnp.kron(eye8, We[i, 2 * H:]), jnp.tile(be[i], 8).reshape(1, 8 * H))

    if i < 2:
      m, aggp = _sc_layer_agg(T, C, senders, receivers)
    else:
      m = _sc_layer_noagg(T, C, senders, receivers)

  dec = pl.pallas_call(
      _dec_body,
      grid=(EP // 1000,),
      in_specs=[pl.BlockSpec((1000, 8 * H), lambda i: (i, 0)),
                _full((8 * H, 8)), _full((1, 8))],
      out_specs=pl.BlockSpec((1000, 8), lambda i: (i, 0)),
      out_shape=jax.ShapeDtypeStruct((EP, 8), _f32),
  )(m, jnp.kron(eye8, Wed), jnp.tile(bed, 8).reshape(1, 8))

  out2 = pl.pallas_call(
      _mask_body,
      in_specs=[_full((625, 512))] * 4 + [_full((1, 1))] * 2,
      out_specs=_full((625, 512)),
      out_shape=jax.ShapeDtypeStruct((625, 512), _f32),
  )(dec.reshape(625, 512), eflat.reshape(625, 512),
    receivers.reshape(625, 512), senders.reshape(625, 512),
    norm, alpha.reshape(1, 1))

  return out2.reshape(NE)


# speedup vs baseline: 1.5297x; 1.1199x over previous
import functools

import jax
import jax.numpy as jnp
from jax import lax
from jax.experimental import pallas as pl
from jax.experimental.pallas import tpu as pltpu
from jax.experimental.pallas import tpu_sc as plsc

NN = 10000
NE = 320000
H = 32
TW = 128
NC, NS = 2, 16
NW = NC * NS
CH = 64
NCHUNK = NE // CH
KMAX = -(-NCHUNK // NW)
NNP = 10240
RPT = NNP // NS
EP = NE // 8
CHP = CH // 8

_f32 = jnp.float32


def _norm_body(e_ref, o_ref):
  o_ref[...] = jnp.max(jnp.abs(e_ref[...])).reshape(1, 1)


def _encode_body(n_ref, wne_ref, bne_ref, ws_ref, wr_ref, hn_ref, t_ref):
  hn = jnp.maximum(
      jnp.dot(n_ref[...], wne_ref[...], preferred_element_type=_f32)
      + bne_ref[...], 0.0)
  hn_ref[...] = hn
  a = jnp.dot(hn, ws_ref[...], preferred_element_type=_f32)
  b = jnp.dot(hn, wr_ref[...], preferred_element_type=_f32)
  z = jnp.zeros((a.shape[0], TW - 2 * H), _f32)
  t_ref[...] = jnp.concatenate([a, b, z], axis=1)


def _c0_body(e_ref, nrm_ref, wee_ref, bee_ref, wse_ref, be_ref, c_ref):
  inv = 1.0 / nrm_ref[0, 0]
  cols = []
  for g in range(8):
    he = jnp.maximum(e_ref[:, g:g + 1] * inv * wee_ref[...] + bee_ref[...], 0.0)
    cols.append(jnp.dot(he, wse_ref[...], preferred_element_type=_f32)
                + be_ref[...])
  c_ref[...] = jnp.concatenate(cols, axis=1)


def _node_body(hn_ref, p0_ref, p1_ref, wn1_ref, wn2_ref, bn_ref, ws_ref, wr_ref,
               hn2_ref, t_ref):
  agg = p0_ref[0, :, :H] + p1_ref[0, :, :H]
  hn2 = jnp.maximum(
      jnp.dot(hn_ref[...], wn1_ref[...], preferred_element_type=_f32)
      + jnp.dot(agg, wn2_ref[...], preferred_element_type=_f32)
      + bn_ref[...], 0.0)
  hn2_ref[...] = hn2
  a = jnp.dot(hn2, ws_ref[...], preferred_element_type=_f32)
  b = jnp.dot(hn2, wr_ref[...], preferred_element_type=_f32)
  z = jnp.zeros((a.shape[0], TW - 2 * H), _f32)
  t_ref[...] = jnp.concatenate([a, b, z], axis=1)


def _cmat_body(m_ref, wse_ref, be_ref, c_ref):
  c_ref[...] = (jnp.dot(m_ref[...], wse_ref[...], preferred_element_type=_f32)
                + be_ref[...])


def _dec_body(m_ref, wed_ref, bed_ref, d_ref):
  d_ref[...] = (jnp.dot(m_ref[...], wed_ref[...], preferred_element_type=_f32)
                + bed_ref[...])


def _mask_body(d_ref, e_ref, r_ref, s_ref, nrm_ref, al_ref, o_ref):
  scale = al_ref[0, 0] * nrm_ref[0, 0]
  val = e_ref[...] + scale * d_ref[...]
  o_ref[...] = jnp.where(r_ref[...] >= s_ref[...], val, 0.0)


def _full(shape):
  return pl.BlockSpec(shape, lambda *a: (0,) * len(shape))


def _sc_layer_body(with_agg, c_packed, t_hbm, c_hbm, s_hbm, r_hbm, *rest):
  if with_agg:
    (m_hbm, aggp_hbm, sidx0, sidx1, ridx0, ridx1, ga, gb, rc0, rc1,
     mc0, mc1, mb, agg_sh, semA, semB, semC, semI, semM, semS) = rest
  else:
    (m_hbm, sidx0, sidx1, ridx0, ridx1, ga, gb, rc0, rc1, mc0, mc1,
     semA, semB, semC, semI, semM) = rest
  sidx = (sidx0, sidx1)
  ridx = (ridx0, ridx1)
  rc = (rc0, rc1)
  mc = (mc0, mc1)

  cid = lax.axis_index("c")
  sid = lax.axis_index("s")
  wid = sid * NC + cid

  if with_agg:
    def zloop(j, _):
      for h in range(TW // 16):
        mb[j, pl.ds(16 * h, 16)] = jnp.zeros((16,), _f32)
      return 0
    lax.fori_loop(0, CH, zloop, 0)

    def zcopy(t, _):
      pltpu.sync_copy(mb, agg_sh.at[pl.ds(sid * RPT + t * CH, CH)])
      return 0
    lax.fori_loop(0, RPT // CH, zcopy, 0)
    plsc.subcore_barrier()

  def issue_fetch(g, p):
    base = g * CH
    ci0 = pltpu.async_copy(s_hbm.at[pl.ds(base, CH)], sidx[p], semI)
    ci1 = pltpu.async_copy(r_hbm.at[pl.ds(base, CH)], ridx[p], semI)
    ci0.wait()
    ci1.wait()
    pltpu.async_copy(t_hbm.at[sidx[p]], ga, semA)
    pltpu.async_copy(t_hbm.at[ridx[p]], gb, semB)
    if c_packed:
      pltpu.async_copy(c_hbm.at[pl.ds(g * CHP, CHP)], rc[p], semC)
    else:
      pltpu.async_copy(c_hbm.at[pl.ds(base, CH)], rc[p], semC)

  def wait_gathers(p):
    pltpu.make_async_copy(t_hbm.at[sidx[p]], ga, semA).wait()
    pltpu.make_async_copy(t_hbm.at[ridx[p]], gb, semB).wait()
    pltpu.make_async_copy(c_hbm.at[pl.ds(0, CHP if c_packed else CH)],
                          rc[p], semC).wait()

  issue_fetch(wid, 0)

  def body(k, p):
    g = k * NW + wid

    @pl.when(g < NCHUNK)
    def _():
      base = g * CH
      wait_gathers(p)
      if with_agg:
        @pl.when(k > 0)
        def _():
          pltpu.make_async_copy(mb, agg_sh.at[ridx[p]], semS).wait()

      @plsc.parallel_loop(0, CH, 1, unroll=8)
      def comp(j):
        jq = j // 8
        jo = (j % 8) * H
        for h in (0, 16):
          if c_packed:
            cv = rc[p][jq, pl.ds(pl.multiple_of(jo + h, 16), 16)]
          else:
            cv = rc[p][j, pl.ds(h, 16)]
          v = ga[j, pl.ds(h, 16)] + gb[j, pl.ds(H + h, 16)] + cv
          v = jnp.maximum(v, 0.0)
          mc[p][jq, pl.ds(pl.multiple_of(jo + h, 16), 16)] = v
          if with_agg:
            mb[j, pl.ds(h, 16)] = v

      pltpu.async_copy(mc[p], m_hbm.at[pl.ds(g * CHP, CHP)], semM)
      if with_agg:
        pltpu.async_copy(mb, agg_sh.at[ridx[p]], semS, add=True)

      g1 = g + NW

      @pl.when(g1 < NCHUNK)
      def _():
        @pl.when(k > 0)
        def _():
          pltpu.make_async_copy(mc[1 - p], m_hbm.at[pl.ds(0, CHP)], semM).wait()
        issue_fetch(g1, 1 - p)

  def two(kk, _):
    body(2 * kk, 0)
    body(2 * kk + 1, 1)
    return 0
  lax.fori_loop(0, (KMAX + 1) // 2, two, 0)

  pltpu.make_async_copy(mc0, m_hbm.at[pl.ds(0, CHP)], semM).wait()
  pltpu.make_async_copy(mc0, m_hbm.at[pl.ds(0, CHP)], semM).wait()
  if with_agg:
    pltpu.make_async_copy(mb, agg_sh.at[ridx0], semS).wait()
    plsc.subcore_barrier()
    r0 = sid * RPT
    pltpu.sync_copy(agg_sh.at[pl.ds(r0, RPT)], aggp_hbm.at[cid, pl.ds(r0, RPT)])


def _make_sc_layer(with_agg, c_packed):
  mesh = plsc.VectorSubcoreMesh(
      core_axis_name="c", subcore_axis_name="s", num_cores=NC, num_subcores=NS)
  if with_agg:
    out_type = (jax.ShapeDtypeStruct((EP, 8 * H), _f32),
                jax.ShapeDtypeStruct((NC, NNP, TW), _f32))
  else:
    out_type = jax.ShapeDtypeStruct((EP, 8 * H), _f32)
  cshape = (CHP, 8 * H) if c_packed else (CH, H)
  scratch = [
      pltpu.VMEM((CH,), jnp.int32),
      pltpu.VMEM((CH,), jnp.int32),
      pltpu.VMEM((CH,), jnp.int32),
      pltpu.VMEM((CH,), jnp.int32),
      pltpu.VMEM((CH, TW), _f32),
      pltpu.VMEM((CH, TW), _f32),
      pltpu.VMEM(cshape, _f32),
      pltpu.VMEM(cshape, _f32),
      pltpu.VMEM((CHP, 8 * H), _f32),
      pltpu.VMEM((CHP, 8 * H), _f32),
  ]
  if with_agg:
    scratch.append(pltpu.VMEM((CH, TW), _f32))
    scratch.append(pltpu.VMEM_SHARED((NNP, TW), _f32))
  nsem = 6 if with_agg else 5
  scratch += [pltpu.SemaphoreType.DMA] * nsem
  return pl.kernel(
      functools.partial(_sc_layer_body, with_agg, c_packed),
      out_type=out_type, mesh=mesh, scratch_types=scratch,
      name="sc_layer_agg" if with_agg else "sc_layer")


_sc_layer_agg0 = _make_sc_layer(True, False)
_sc_layer_agg = _make_sc_layer(True, True)
_sc_layer_noagg = _make_sc_layer(False, True)


def kernel(nodes, edges, receivers, senders, Wne, bne, Wee, bee, We, be, Wn,
           bn, Wed, bed, alpha):
  eflat = edges.reshape(NE)
  norm = pl.pallas_call(
      _norm_body,
      out_shape=jax.ShapeDtypeStruct((1, 1), _f32),
  )(eflat.reshape(2500, 128))

  NB = 1000
  hn, T = pl.pallas_call(
      _encode_body,
      grid=(NN // NB,),
      in_specs=[pl.BlockSpec((NB, 128), lambda i: (i, 0)),
                _full((128, H)), _full((1, H)), _full((H, H)), _full((H, H))],
      out_specs=[pl.BlockSpec((NB, H), lambda i: (i, 0)),
                 pl.BlockSpec((NB, TW), lambda i: (i, 0))],
      out_shape=[jax.ShapeDtypeStruct((NN, H), _f32),
                 jax.ShapeDtypeStruct((NN, TW), _f32)],
  )(nodes, Wne, bne.reshape(1, H), We[0, :H], We[0, H:2 * H])

  EB = 4000
  C = pl.pallas_call(
      _c0_body,
      grid=(EP // 1000,),
      in_specs=[pl.BlockSpec((1000, 8), lambda i: (i, 0)),
                _full((1, 1)), _full((1, H)), _full((1, H)),
                _full((H, H)), _full((1, H))],
      out_specs=pl.BlockSpec((1000, 8 * H), lambda i: (i, 0)),
      out_shape=jax.ShapeDtypeStruct((EP, 8 * H), _f32),
  )(eflat.reshape(EP, 8), norm, Wee, bee.reshape(1, H), We[0, 2 * H:],
    be[0].reshape(1, H))

  m, aggp = _sc_layer_agg(T, C, senders, receivers)

  eye8 = jnp.eye(8, dtype=_f32)
  for i in (1, 2):
    hn, T = pl.pallas_call(
        _node_body,
        grid=(NN // NB,),
        in_specs=[pl.BlockSpec((NB, H), lambda i: (i, 0)),
                  pl.BlockSpec((1, NB, TW), lambda i: (0, i, 0)),
                  pl.BlockSpec((1, NB, TW), lambda i: (1, i, 0)),
                  _full((H, H)), _full((H, H)), _full((1, H)),
                  _full((H, H)), _full((H, H))],
        out_specs=[pl.BlockSpec((NB, H), lambda i: (i, 0)),
                   pl.BlockSpec((NB, TW), lambda i: (i, 0))],
        out_shape=[jax.ShapeDtypeStruct((NN, H), _f32),
                   jax.ShapeDtypeStruct((NN, TW), _f32)],
    )(hn, aggp, aggp, Wn[i - 1, :H], Wn[i - 1, H:],
      bn[i - 1].reshape(1, H), We[i, :H], We[i, H:2 * H])

    C = pl.pallas_call(
        _cmat_body,
        grid=(EP // 1000,),
        in_specs=[pl.BlockSpec((1000, 8 * H), lambda i: (i, 0)),
                  _full((8 * H, 8 * H)), _full((1, 8 * H))],
        out_specs=pl.BlockSpec((1000, 8 * H), lambda i: (i, 0)),
        out_shape=jax.ShapeDtypeStruct((EP, 8 * H), _f32),
    )(m, jnp.kron(eye8, We[i, 2 * H:]), jnp.tile(be[i], 8).reshape(1, 8 * H))

    if i < 2:
      m, aggp = _sc_layer_agg(T, C, senders, receivers)
    else:
      m = _sc_layer_noagg(T, C, senders, receivers)

  dec = pl.pallas_call(
      _dec_body,
      grid=(EP // 1000,),
      in_specs=[pl.BlockSpec((1000, 8 * H), lambda i: (i, 0)),
                _full((8 * H, 8)), _full((1, 8))],
      out_specs=pl.BlockSpec((1000, 8), lambda i: (i, 0)),
      out_shape=jax.ShapeDtypeStruct((EP, 8), _f32),
  )(m, jnp.kron(eye8, Wed), jnp.tile(bed, 8).reshape(1, 8))

  out2 = pl.pallas_call(
      _mask_body,
      in_specs=[_full((625, 512))] * 4 + [_full((1, 1))] * 2,
      out_specs=_full((625, 512)),
      out_shape=jax.ShapeDtypeStruct((625, 512), _f32),
  )(dec.reshape(625, 512), eflat.reshape(625, 512),
    receivers.reshape(625, 512), senders.reshape(625, 512),
    norm, alpha.reshape(1, 1))

  return out2.reshape(NE)
